# baseline (speedup 1.0000x reference)
import jax
import jax.numpy as jnp
from jax.experimental import pallas as pl

B, N, F_NODE, HG, OBS_DIM, HID, ACT = 4096, 100, 16, 16, 32, 64, 5
NPAD = 104
PK = 8
RPB = NPAD // PK
BB = 128
R = BB * RPB

_F32 = jnp.float32
_BF16 = jnp.bfloat16
_HI = jax.lax.Precision.HIGHEST


def _body(x2_ref, wl_ref, aid_ref, obs_ref,
          w8in_ref, b8_ref, w8msg_ref, wupd_ref,
          w1a_ref, w1b_ref, b1_ref, w2_ref, b2_ref, wq_ref, bq_ref,
          q_ref):
    x = x2_ref[...]
    h = jnp.maximum(
        jnp.dot(x, w8in_ref[...], preferred_element_type=_F32)
        + b8_ref[...], 0.0)
    msg = jnp.maximum(
        jnp.dot(h.astype(_BF16), w8msg_ref[...], preferred_element_type=_F32),
        0.0)
    prod = msg.astype(_BF16).astype(_F32) * wl_ref[...].astype(_F32)

    rows_i = jax.lax.broadcasted_iota(jnp.int32, (BB, R), 1)
    b_i = jax.lax.broadcasted_iota(jnp.int32, (BB, R), 0)
    S = jnp.where((rows_i >= b_i * RPB) & (rows_i < (b_i + 1) * RPB),
                  1.0, 0.0).astype(_F32)
    rows_t = jax.lax.broadcasted_iota(jnp.int32, (R, BB), 0)
    b_t = jax.lax.broadcasted_iota(jnp.int32, (R, BB), 1)
    ST = jnp.where((rows_t >= b_t * RPB) & (rows_t < (b_t + 1) * RPB),
                   1.0, 0.0).astype(_F32)

    b_col = jax.lax.broadcasted_iota(jnp.int32, (BB, 1), 0).astype(_F32)
    bidx = jnp.dot(ST, b_col, precision=_HI,
                   preferred_element_type=_F32).astype(jnp.int32)
    idxrow = jnp.dot(ST, aid_ref[...], precision=_HI,
                     preferred_element_type=_F32).astype(jnp.int32)
    rows_c = jax.lax.broadcasted_iota(jnp.int32, (R, 1), 0)
    rm = rows_c - RPB * bidx
    lane = jax.lax.broadcasted_iota(jnp.int32, (R, 128), 1)
    nodeid = PK * rm + (lane >> 4)
    ohl = jnp.where(nodeid == idxrow, 1.0, 0.0).astype(_F32)

    agg128 = jnp.dot(S, prod, precision=_HI, preferred_element_type=_F32)
    hsel128 = jnp.dot(S, h * ohl, precision=_HI, preferred_element_type=_F32)

    lane_j = jax.lax.broadcasted_iota(jnp.int32, (128, HG), 0)
    lane_f = jax.lax.broadcasted_iota(jnp.int32, (128, HG), 1)
    F = jnp.where((lane_j & 15) == lane_f, 1.0, 0.0).astype(_F32)
    agg = jnp.dot(agg128, F, precision=_HI, preferred_element_type=_F32)
    hsel = jnp.dot(hsel128, F, precision=_HI, preferred_element_type=_F32)

    gnn = jnp.maximum(
        hsel + jnp.dot(agg.astype(_BF16), wupd_ref[...],
                       preferred_element_type=_F32), 0.0)
    x1 = jnp.maximum(
        jnp.dot(obs_ref[...].astype(_BF16), w1a_ref[...],
                preferred_element_type=_F32)
        + jnp.dot(gnn.astype(_BF16), w1b_ref[...],
                  preferred_element_type=_F32)
        + b1_ref[...], 0.0)
    x2m = jnp.maximum(
        jnp.dot(x1.astype(_BF16), w2_ref[...], preferred_element_type=_F32)
        + b2_ref[...], 0.0)
    q_ref[...] = (jnp.dot(x2m.astype(_BF16), wq_ref[...],
                          preferred_element_type=_F32) + bq_ref[...])


def kernel(obs, rnn_states, node_obs, adj, agent_id, W_in, b_in, W_msg,
           W_upd, W1, b1, W2, b2, Wq, bq):
    idx = agent_id[:, 0].astype(jnp.int32)
    adj_row = jnp.take_along_axis(adj, idx[:, None, None], axis=1)[:, 0, :]
    w = adj_row * (adj_row > 0).astype(adj_row.dtype)
    wp = jnp.pad(w, ((0, 0), (0, NPAD - N)))
    wl = jnp.repeat(wp.reshape(B * RPB, PK), HG, axis=1).astype(_BF16)

    x2 = jnp.pad(node_obs, ((0, 0), (0, NPAD - N), (0, 0))
                 ).reshape(B * RPB, PK * F_NODE).astype(_BF16)

    eye8 = jnp.eye(PK, dtype=_F32)
    w8in = jnp.kron(eye8, W_in).astype(_BF16)
    w8msg = jnp.kron(eye8, W_msg).astype(_BF16)
    b8 = jnp.tile(b_in, PK)[None, :]
    aidf = idx.astype(_F32)[:, None]

    full = lambda shape: pl.BlockSpec(shape, lambda i: (0,) * len(shape))
    q = pl.pallas_call(
        _body,
        grid=(B // BB,),
        in_specs=[
            pl.BlockSpec((R, 128), lambda i: (i, 0)),
            pl.BlockSpec((R, 128), lambda i: (i, 0)),
            pl.BlockSpec((BB, 1), lambda i: (i, 0)),
            pl.BlockSpec((BB, OBS_DIM), lambda i: (i, 0)),
            full((128, 128)),
            full((1, 128)),
            full((128, 128)),
            full((HG, HG)),
            full((OBS_DIM, HID)),
            full((HG, HID)),
            full((1, HID)),
            full((HID, HID)),
            full((1, HID)),
            full((HID, ACT)),
            full((1, ACT)),
        ],
        out_specs=pl.BlockSpec((BB, ACT), lambda i: (i, 0)),
        out_shape=jax.ShapeDtypeStruct((B, ACT), _F32),
    )(x2, wl, aidf, obs,
      w8in, b8, w8msg, W_upd.astype(_BF16),
      W1[:OBS_DIM].astype(_BF16), W1[OBS_DIM:].astype(_BF16), b1[None, :],
      W2.astype(_BF16), b2[None, :], Wq.astype(_BF16), bq[None, :])
    return (q, rnn_states)


# speedup vs baseline: 1.6131x; 1.6131x over previous
import jax
import jax.numpy as jnp
from jax.experimental import pallas as pl
from jax.experimental.pallas import tpu as pltpu

B, N, F_NODE, HG, OBS_DIM, HID, ACT = 4096, 100, 16, 16, 32, 64, 5
NPAD = 104
PK = 8
RPB = NPAD // PK
BB = 128
R = BB * RPB

_F32 = jnp.float32
_BF16 = jnp.bfloat16
_HI = jax.lax.Precision.HIGHEST


def _body(x2_ref, wl_ref, aid_ref, obs_ref,
          w8in_ref, b8_ref, w8msg_ref, wupd_ref,
          w1a_ref, w1b_ref, b1_ref, w2_ref, b2_ref, wq_ref, bq_ref,
          q_ref, sf_ref, sb_ref, st_ref, nid_ref):

    @pl.when(pl.program_id(0) == 0)
    def _init():
        rows_i = jax.lax.broadcasted_iota(jnp.int32, (BB, R), 1)
        b_i = jax.lax.broadcasted_iota(jnp.int32, (BB, R), 0)
        band = (rows_i >= b_i * RPB) & (rows_i < (b_i + 1) * RPB)
        sf_ref[...] = jnp.where(band, 1.0, 0.0)
        sb_ref[...] = jnp.where(band, 1.0, 0.0).astype(_BF16)
        rows_t = jax.lax.broadcasted_iota(jnp.int32, (R, BB), 0)
        b_t = jax.lax.broadcasted_iota(jnp.int32, (R, BB), 1)
        st_ref[...] = jnp.where(
            (rows_t >= b_t * RPB) & (rows_t < (b_t + 1) * RPB),
            1.0, 0.0).astype(_BF16)
        rows = jax.lax.broadcasted_iota(jnp.int32, (R, 128), 0)
        lane = jax.lax.broadcasted_iota(jnp.int32, (R, 128), 1)
        div13 = (rows * 5042) >> 16
        nid_ref[...] = PK * (rows - RPB * div13) + (lane >> 4)

    x = x2_ref[...]
    h = jnp.maximum(
        jnp.dot(x, w8in_ref[...], preferred_element_type=_F32)
        + b8_ref[...], 0.0)
    msg = jnp.maximum(
        jnp.dot(h.astype(_BF16), w8msg_ref[...], preferred_element_type=_F32),
        0.0)
    prod = msg.astype(_BF16) * wl_ref[...]

    agg128 = jnp.dot(sb_ref[...], prod, preferred_element_type=_F32)

    idxrow = jnp.dot(st_ref[...], aid_ref[...],
                     preferred_element_type=_F32).astype(jnp.int32)
    ohl = jnp.where(nid_ref[...] == idxrow, 1.0, 0.0)
    hsel128 = jnp.dot(sf_ref[...], h * ohl, precision=_HI,
                      preferred_element_type=_F32)

    lane_j = jax.lax.broadcasted_iota(jnp.int32, (128, HG), 0)
    lane_f = jax.lax.broadcasted_iota(jnp.int32, (128, HG), 1)
    fold = jnp.where((lane_j & 15) == lane_f, 1.0, 0.0)
    agg = jnp.dot(agg128, fold, precision=_HI, preferred_element_type=_F32)
    hsel = jnp.dot(hsel128, fold, precision=_HI, preferred_element_type=_F32)

    gnn = jnp.maximum(
        hsel + jnp.dot(agg.astype(_BF16), wupd_ref[...],
                       preferred_element_type=_F32), 0.0)
    x1 = jnp.maximum(
        jnp.dot(obs_ref[...].astype(_BF16), w1a_ref[...],
                preferred_element_type=_F32)
        + jnp.dot(gnn.astype(_BF16), w1b_ref[...],
                  preferred_element_type=_F32)
        + b1_ref[...], 0.0)
    x2m = jnp.maximum(
        jnp.dot(x1.astype(_BF16), w2_ref[...], preferred_element_type=_F32)
        + b2_ref[...], 0.0)
    q_ref[...] = (jnp.dot(x2m.astype(_BF16), wq_ref[...],
                          preferred_element_type=_F32) + bq_ref[...])


def kernel(obs, rnn_states, node_obs, adj, agent_id, W_in, b_in, W_msg,
           W_upd, W1, b1, W2, b2, Wq, bq):
    idx = agent_id[:, 0].astype(jnp.int32)
    adj_row = jnp.take_along_axis(adj, idx[:, None, None], axis=1)[:, 0, :]
    w = adj_row * (adj_row > 0).astype(adj_row.dtype)
    wp = jnp.pad(w, ((0, 0), (0, NPAD - N)))
    wl = jnp.repeat(wp.reshape(B * RPB, PK), HG, axis=1).astype(_BF16)

    x2 = jnp.pad(node_obs.reshape(B, N * F_NODE).astype(_BF16),
                 ((0, 0), (0, (NPAD - N) * F_NODE))
                 ).reshape(B * RPB, PK * F_NODE)

    eye8 = jnp.eye(PK, dtype=_F32)
    w8in = jnp.kron(eye8, W_in).astype(_BF16)
    w8msg = jnp.kron(eye8, W_msg).astype(_BF16)
    b8 = jnp.tile(b_in, PK)[None, :]
    aidb = idx.astype(_BF16)[:, None]

    full = lambda shape: pl.BlockSpec(shape, lambda i: (0,) * len(shape))
    q = pl.pallas_call(
        _body,
        grid=(B // BB,),
        in_specs=[
            pl.BlockSpec((R, 128), lambda i: (i, 0)),
            pl.BlockSpec((R, 128), lambda i: (i, 0)),
            pl.BlockSpec((BB, 1), lambda i: (i, 0)),
            pl.BlockSpec((BB, OBS_DIM), lambda i: (i, 0)),
            full((128, 128)),
            full((1, 128)),
            full((128, 128)),
            full((HG, HG)),
            full((OBS_DIM, HID)),
            full((HG, HID)),
            full((1, HID)),
            full((HID, HID)),
            full((1, HID)),
            full((HID, ACT)),
            full((1, ACT)),
        ],
        out_specs=pl.BlockSpec((BB, ACT), lambda i: (i, 0)),
        out_shape=jax.ShapeDtypeStruct((B, ACT), _F32),
        scratch_shapes=[
            pltpu.VMEM((BB, R), _F32),
            pltpu.VMEM((BB, R), _BF16),
            pltpu.VMEM((R, BB), _BF16),
            pltpu.VMEM((R, 128), jnp.int32),
        ],
    )(x2, wl, aidb, obs,
      w8in, b8, w8msg, W_upd.astype(_BF16),
      W1[:OBS_DIM].astype(_BF16), W1[OBS_DIM:].astype(_BF16), b1[None, :],
      W2.astype(_BF16), b2[None, :], Wq.astype(_BF16), bq[None, :])
    return (q, rnn_states)


# speedup vs baseline: 1.7111x; 1.0608x over previous
import jax
import jax.numpy as jnp
from jax.experimental import pallas as pl
from jax.experimental.pallas import tpu as pltpu

B, N, F_NODE, HG, OBS_DIM, HID, ACT = 4096, 100, 16, 16, 32, 64, 5
NPAD = 104
PK = 8
RPB = NPAD // PK
BB = 256
R = BB * RPB

_F32 = jnp.float32
_BF16 = jnp.bfloat16
_HI = jax.lax.Precision.HIGHEST


def _body(x2_ref, wl_ref, aid_ref, obs_ref,
          w8in_ref, b8_ref, w8msg_ref, wupd_ref,
          w1a_ref, w1b_ref, b1_ref, w2_ref, b2_ref, wq_ref, bq_ref,
          q_ref, sb_ref, st_ref, nid_ref):

    @pl.when(pl.program_id(0) == 0)
    def _init():
        rows_i = jax.lax.broadcasted_iota(jnp.int32, (BB, R), 1)
        b_i = jax.lax.broadcasted_iota(jnp.int32, (BB, R), 0)
        band = (rows_i >= b_i * RPB) & (rows_i < (b_i + 1) * RPB)
        sb_ref[...] = jnp.where(band, 1.0, 0.0).astype(_BF16)
        rows_t = jax.lax.broadcasted_iota(jnp.int32, (R, BB), 0)
        b_t = jax.lax.broadcasted_iota(jnp.int32, (R, BB), 1)
        st_ref[...] = jnp.where(
            (rows_t >= b_t * RPB) & (rows_t < (b_t + 1) * RPB),
            1.0, 0.0).astype(_BF16)
        rows = jax.lax.broadcasted_iota(jnp.int32, (R, 128), 0)
        lane = jax.lax.broadcasted_iota(jnp.int32, (R, 128), 1)
        div13 = (rows * 5042) >> 16
        nid_ref[...] = PK * (rows - RPB * div13) + (lane >> 4)

    x = x2_ref[...]
    h = jnp.maximum(
        jnp.dot(x, w8in_ref[...], preferred_element_type=_F32)
        + b8_ref[...], 0.0)
    msg = jnp.maximum(
        jnp.dot(h.astype(_BF16), w8msg_ref[...], preferred_element_type=_F32),
        0.0)
    prod = msg.astype(_BF16) * wl_ref[...]

    agg128 = jnp.dot(sb_ref[...], prod, preferred_element_type=_F32)

    idxrow = jnp.dot(st_ref[...], aid_ref[...],
                     preferred_element_type=_F32).astype(jnp.int32)
    ohl = jnp.where(nid_ref[...] == idxrow, 1.0, 0.0)
    m = h * ohl
    m_hi = m.astype(_BF16)
    m_lo = (m - m_hi.astype(_F32)).astype(_BF16)
    hsel128 = (jnp.dot(sb_ref[...], m_hi, preferred_element_type=_F32)
               + jnp.dot(sb_ref[...], m_lo, preferred_element_type=_F32))

    lane_j = jax.lax.broadcasted_iota(jnp.int32, (128, HG), 0)
    lane_f = jax.lax.broadcasted_iota(jnp.int32, (128, HG), 1)
    fold = jnp.where((lane_j & 15) == lane_f, 1.0, 0.0)
    agg = jnp.dot(agg128, fold, precision=_HI, preferred_element_type=_F32)
    hsel = jnp.dot(hsel128, fold, precision=_HI, preferred_element_type=_F32)

    gnn = jnp.maximum(
        hsel + jnp.dot(agg.astype(_BF16), wupd_ref[...],
                       preferred_element_type=_F32), 0.0)
    x1 = jnp.maximum(
        jnp.dot(obs_ref[...].astype(_BF16), w1a_ref[...],
                preferred_element_type=_F32)
        + jnp.dot(gnn.astype(_BF16), w1b_ref[...],
                  preferred_element_type=_F32)
        + b1_ref[...], 0.0)
    x2m = jnp.maximum(
        jnp.dot(x1.astype(_BF16), w2_ref[...], preferred_element_type=_F32)
        + b2_ref[...], 0.0)
    q_ref[...] = (jnp.dot(x2m.astype(_BF16), wq_ref[...],
                          preferred_element_type=_F32) + bq_ref[...])


def kernel(obs, rnn_states, node_obs, adj, agent_id, W_in, b_in, W_msg,
           W_upd, W1, b1, W2, b2, Wq, bq):
    idx = agent_id[:, 0].astype(jnp.int32)
    adj_row = jnp.take_along_axis(adj, idx[:, None, None], axis=1)[:, 0, :]
    w = adj_row * (adj_row > 0).astype(adj_row.dtype)
    wp = jnp.pad(w, ((0, 0), (0, NPAD - N)))
    wl = jnp.repeat(wp.reshape(B * RPB, PK), HG, axis=1).astype(_BF16)

    x2 = jnp.pad(node_obs.reshape(B, N * F_NODE).astype(_BF16),
                 ((0, 0), (0, (NPAD - N) * F_NODE))
                 ).reshape(B * RPB, PK * F_NODE)

    eye8 = jnp.eye(PK, dtype=_F32)
    w8in = jnp.kron(eye8, W_in).astype(_BF16)
    w8msg = jnp.kron(eye8, W_msg).astype(_BF16)
    b8 = jnp.tile(b_in, PK)[None, :]
    aidb = idx.astype(_BF16)[:, None]

    full = lambda shape: pl.BlockSpec(shape, lambda i: (0,) * len(shape))
    q = pl.pallas_call(
        _body,
        grid=(B // BB,),
        in_specs=[
            pl.BlockSpec((R, 128), lambda i: (i, 0)),
            pl.BlockSpec((R, 128), lambda i: (i, 0)),
            pl.BlockSpec((BB, 1), lambda i: (i, 0)),
            pl.BlockSpec((BB, OBS_DIM), lambda i: (i, 0)),
            full((128, 128)),
            full((1, 128)),
            full((128, 128)),
            full((HG, HG)),
            full((OBS_DIM, HID)),
            full((HG, HID)),
            full((1, HID)),
            full((HID, HID)),
            full((1, HID)),
            full((HID, ACT)),
            full((1, ACT)),
        ],
        out_specs=pl.BlockSpec((BB, ACT), lambda i: (i, 0)),
        out_shape=jax.ShapeDtypeStruct((B, ACT), _F32),
        scratch_shapes=[
            pltpu.VMEM((BB, R), _BF16),
            pltpu.VMEM((R, BB), _BF16),
            pltpu.VMEM((R, 128), jnp.int32),
        ],
    )(x2, wl, aidb, obs,
      w8in, b8, w8msg, W_upd.astype(_BF16),
      W1[:OBS_DIM].astype(_BF16), W1[OBS_DIM:].astype(_BF16), b1[None, :],
      W2.astype(_BF16), b2[None, :], Wq.astype(_BF16), bq[None, :])
    return (q, rnn_states)


# speedup vs baseline: 1.7834x; 1.0423x over previous
import jax
import jax.numpy as jnp
from jax.experimental import pallas as pl
from jax.experimental.pallas import tpu as pltpu

B, N, F_NODE, HG, OBS_DIM, HID, ACT = 4096, 100, 16, 16, 32, 64, 5
NPAD = 104
PK = 8
RPB = NPAD // PK
BB = 256
R = BB * RPB

_F32 = jnp.float32
_BF16 = jnp.bfloat16
_HI = jax.lax.Precision.HIGHEST


def _body(x2_ref, wl_ref, aid_ref, obs_ref,
          w8in_ref, b8_ref, w8msg_ref, wupd_ref,
          w1a_ref, w1b_ref, b1_ref, w2_ref, b2_ref, wq_ref, bq_ref,
          q_ref, sb_ref, st_ref, nid_ref):

    @pl.when(pl.program_id(0) == 0)
    def _init():
        rows_i = jax.lax.broadcasted_iota(jnp.int32, (BB, R), 1)
        b_i = jax.lax.broadcasted_iota(jnp.int32, (BB, R), 0)
        band = (rows_i >= b_i * RPB) & (rows_i < (b_i + 1) * RPB)
        sb_ref[...] = jnp.where(band, 1.0, 0.0).astype(_BF16)
        rows_t = jax.lax.broadcasted_iota(jnp.int32, (R, BB), 0)
        b_t = jax.lax.broadcasted_iota(jnp.int32, (R, BB), 1)
        st_ref[...] = jnp.where(
            (rows_t >= b_t * RPB) & (rows_t < (b_t + 1) * RPB),
            1.0, 0.0).astype(_BF16)
        rows = jax.lax.broadcasted_iota(jnp.int32, (R, 128), 0)
        lane = jax.lax.broadcasted_iota(jnp.int32, (R, 128), 1)
        div13 = (rows * 5042) >> 16
        nid_ref[...] = PK * (rows - RPB * div13) + (lane >> 4)

    x = x2_ref[...]
    h = jnp.maximum(
        jnp.dot(x, w8in_ref[...], preferred_element_type=_F32)
        + b8_ref[...], 0.0)
    msg = jnp.maximum(
        jnp.dot(h.astype(_BF16), w8msg_ref[...], preferred_element_type=_F32),
        0.0)
    prod = msg.astype(_BF16) * wl_ref[...]

    agg128 = jnp.dot(sb_ref[...], prod, preferred_element_type=_F32)

    idxrow = jnp.dot(st_ref[...], aid_ref[...],
                     preferred_element_type=_F32).astype(jnp.int32)
    ohl = jnp.where(nid_ref[...] == idxrow, 1.0, 0.0)
    m = (h * ohl).astype(_BF16)
    hsel128 = jnp.dot(sb_ref[...], m, preferred_element_type=_F32)

    lane_j = jax.lax.broadcasted_iota(jnp.int32, (128, HG), 0)
    lane_f = jax.lax.broadcasted_iota(jnp.int32, (128, HG), 1)
    fold = jnp.where((lane_j & 15) == lane_f, 1.0, 0.0).astype(_BF16)
    agg = jnp.dot(agg128.astype(_BF16), fold, preferred_element_type=_F32)
    hsel = jnp.dot(hsel128.astype(_BF16), fold, preferred_element_type=_F32)

    gnn = jnp.maximum(
        hsel + jnp.dot(agg.astype(_BF16), wupd_ref[...],
                       preferred_element_type=_F32), 0.0)
    x1 = jnp.maximum(
        jnp.dot(obs_ref[...].astype(_BF16), w1a_ref[...],
                preferred_element_type=_F32)
        + jnp.dot(gnn.astype(_BF16), w1b_ref[...],
                  preferred_element_type=_F32)
        + b1_ref[...], 0.0)
    x2m = jnp.maximum(
        jnp.dot(x1.astype(_BF16), w2_ref[...], preferred_element_type=_F32)
        + b2_ref[...], 0.0)
    q_ref[...] = (jnp.dot(x2m.astype(_BF16), wq_ref[...],
                          preferred_element_type=_F32) + bq_ref[...])


def kernel(obs, rnn_states, node_obs, adj, agent_id, W_in, b_in, W_msg,
           W_upd, W1, b1, W2, b2, Wq, bq):
    idx = agent_id[:, 0].astype(jnp.int32)
    adj_row = jnp.take_along_axis(adj, idx[:, None, None], axis=1)[:, 0, :]
    w = adj_row * (adj_row > 0).astype(adj_row.dtype)
    wp = jnp.pad(w, ((0, 0), (0, NPAD - N)))
    wl = jnp.repeat(wp.reshape(B * RPB, PK), HG, axis=1).astype(_BF16)

    x2 = jnp.pad(node_obs.reshape(B, N * F_NODE).astype(_BF16),
                 ((0, 0), (0, (NPAD - N) * F_NODE))
                 ).reshape(B * RPB, PK * F_NODE)

    eye8 = jnp.eye(PK, dtype=_F32)
    w8in = jnp.kron(eye8, W_in).astype(_BF16)
    w8msg = jnp.kron(eye8, W_msg).astype(_BF16)
    b8 = jnp.tile(b_in, PK)[None, :]
    aidb = idx.astype(_BF16)[:, None]

    full = lambda shape: pl.BlockSpec(shape, lambda i: (0,) * len(shape))
    q = pl.pallas_call(
        _body,
        grid=(B // BB,),
        in_specs=[
            pl.BlockSpec((R, 128), lambda i: (i, 0)),
            pl.BlockSpec((R, 128), lambda i: (i, 0)),
            pl.BlockSpec((BB, 1), lambda i: (i, 0)),
            pl.BlockSpec((BB, OBS_DIM), lambda i: (i, 0)),
            full((128, 128)),
            full((1, 128)),
            full((128, 128)),
            full((HG, HG)),
            full((OBS_DIM, HID)),
            full((HG, HID)),
            full((1, HID)),
            full((HID, HID)),
            full((1, HID)),
            full((HID, ACT)),
            full((1, ACT)),
        ],
        out_specs=pl.BlockSpec((BB, ACT), lambda i: (i, 0)),
        out_shape=jax.ShapeDtypeStruct((B, ACT), _F32),
        scratch_shapes=[
            pltpu.VMEM((BB, R), _BF16),
            pltpu.VMEM((R, BB), _BF16),
            pltpu.VMEM((R, 128), jnp.int32),
        ],
    )(x2, wl, aidb, obs,
      w8in, b8, w8msg, W_upd.astype(_BF16),
      W1[:OBS_DIM].astype(_BF16), W1[OBS_DIM:].astype(_BF16), b1[None, :],
      W2.astype(_BF16), b2[None, :], Wq.astype(_BF16), bq[None, :])
    return (q, rnn_states)
